# baseline (speedup 1.0000x reference)
import functools

import jax
import jax.numpy as jnp
from jax.experimental import pallas as pl
from jax.experimental.pallas import tpu as pltpu

_WEIGHT = 0.01


def _chamfer_body(x_ref, y_ref, o_ref, rowmin_ref, colmin_ref, *, n_blocks,
                  m_blocks, tile_m, n, m):
    nb = pl.program_id(1)
    mb = pl.program_id(2)

    x = x_ref[0]
    y = y_ref[0]
    x2 = jnp.sum(x * x, axis=1)
    y2 = jnp.sum(y * y, axis=1)
    xy = jax.lax.dot_general(
        x, y, (((1,), (1,)), ((), ())), preferred_element_type=jnp.float32)
    d = jnp.maximum(x2[:, None] + y2[None, :] - 2.0 * xy, 0.0)
    brow = jnp.min(d, axis=1)
    bcol = jnp.min(d, axis=0)

    @pl.when(jnp.logical_and(nb == 0, mb == 0))
    def _():
        o_ref[0, 0, :] = jnp.zeros((128,), jnp.float32)

    @pl.when(mb == 0)
    def _():
        rowmin_ref[0, :] = brow

    @pl.when(mb > 0)
    def _():
        rowmin_ref[0, :] = jnp.minimum(rowmin_ref[0, :], brow)

    @pl.when(mb == m_blocks - 1)
    def _():
        o_ref[0, 0, :] += jnp.full((128,), jnp.sum(rowmin_ref[0, :]) * (1.0 / n))

    sl = pl.ds(mb * tile_m, tile_m)

    @pl.when(nb == 0)
    def _():
        colmin_ref[0, sl] = bcol

    @pl.when(nb > 0)
    def _():
        colmin_ref[0, sl] = jnp.minimum(colmin_ref[0, sl], bcol)

    @pl.when(nb == n_blocks - 1)
    def _():
        o_ref[0, 0, :] += jnp.full((128,), jnp.sum(colmin_ref[0, sl]) * (1.0 / m))


def kernel(transformed_source, transformed_target):
    x = transformed_source.astype(jnp.float32)
    y = transformed_target.astype(jnp.float32)
    b, n, d = x.shape
    _, m, _ = y.shape

    tile_n = 1024
    tile_m = 1024
    n_blocks = n // tile_n
    m_blocks = m // tile_m

    body = functools.partial(
        _chamfer_body, n_blocks=n_blocks, m_blocks=m_blocks, tile_m=tile_m,
        n=n, m=m)

    out = pl.pallas_call(
        body,
        grid=(b, n_blocks, m_blocks),
        in_specs=[
            pl.BlockSpec((1, tile_n, d), lambda bi, ni, mi: (bi, ni, 0)),
            pl.BlockSpec((1, tile_m, d), lambda bi, ni, mi: (bi, mi, 0)),
        ],
        out_specs=pl.BlockSpec((1, 1, 128), lambda bi, ni, mi: (bi, 0, 0)),
        out_shape=jax.ShapeDtypeStruct((b, 1, 128), jnp.float32),
        scratch_shapes=[
            pltpu.VMEM((1, tile_n), jnp.float32),
            pltpu.VMEM((1, m), jnp.float32),
        ],
    )(x, y)

    return _WEIGHT * jnp.mean(out[:, 0, 0])


# speedup vs baseline: 1.4176x; 1.4176x over previous
import functools

import jax
import jax.numpy as jnp
from jax.experimental import pallas as pl
from jax.experimental.pallas import tpu as pltpu

_WEIGHT = 0.01


def _chamfer_body(x_ref, ys_ref, o_ref, rowmin_ref, colmin_ref, *, n_blocks,
                  m_blocks, tile_m, n, m):
    nb = pl.program_id(1)
    mb = pl.program_id(2)

    x = x_ref[0]
    ys = ys_ref[0]
    x2 = jnp.sum(x * x, axis=1)
    y2 = 0.25 * jnp.sum(ys * ys, axis=1)
    t = jax.lax.dot_general(
        x, ys, (((1,), (1,)), ((), ())),
        preferred_element_type=jnp.float32)
    brow = jnp.min(t + y2[None, :], axis=1)
    bcol = jnp.min(t + x2[:, None], axis=0)

    @pl.when(jnp.logical_and(nb == 0, mb == 0))
    def _():
        o_ref[0, 0, :] = jnp.zeros((128,), jnp.float32)

    @pl.when(mb == 0)
    def _():
        rowmin_ref[0, :] = brow

    @pl.when(mb > 0)
    def _():
        rowmin_ref[0, :] = jnp.minimum(rowmin_ref[0, :], brow)

    @pl.when(mb == m_blocks - 1)
    def _():
        cham_x = jnp.maximum(rowmin_ref[0, :] + x2, 0.0)
        o_ref[0, 0, :] += jnp.full((128,), jnp.sum(cham_x) * (1.0 / n))

    sl = pl.ds(mb * tile_m, tile_m)

    @pl.when(nb == 0)
    def _():
        colmin_ref[0, sl] = bcol

    @pl.when(nb > 0)
    def _():
        colmin_ref[0, sl] = jnp.minimum(colmin_ref[0, sl], bcol)

    @pl.when(nb == n_blocks - 1)
    def _():
        cham_y = jnp.maximum(colmin_ref[0, sl] + y2, 0.0)
        o_ref[0, 0, :] += jnp.full((128,), jnp.sum(cham_y) * (1.0 / m))


def kernel(transformed_source, transformed_target):
    x = transformed_source.astype(jnp.float32)
    y = transformed_target.astype(jnp.float32)
    b, n, d = x.shape
    _, m, _ = y.shape
    ys = -2.0 * y

    tile_n = 2048
    tile_m = 2048
    n_blocks = n // tile_n
    m_blocks = m // tile_m

    body = functools.partial(
        _chamfer_body, n_blocks=n_blocks, m_blocks=m_blocks, tile_m=tile_m,
        n=n, m=m)

    out = pl.pallas_call(
        body,
        grid=(b, n_blocks, m_blocks),
        in_specs=[
            pl.BlockSpec((1, tile_n, d), lambda bi, ni, mi: (bi, ni, 0)),
            pl.BlockSpec((1, tile_m, d), lambda bi, ni, mi: (bi, mi, 0)),
        ],
        out_specs=pl.BlockSpec((1, 1, 128), lambda bi, ni, mi: (bi, 0, 0)),
        out_shape=jax.ShapeDtypeStruct((b, 1, 128), jnp.float32),
        scratch_shapes=[
            pltpu.VMEM((1, tile_n), jnp.float32),
            pltpu.VMEM((1, m), jnp.float32),
        ],
    )(x, ys)

    return _WEIGHT * jnp.mean(out[:, 0, 0])
